# baseline (speedup 1.0000x reference)
import functools

import jax
import jax.numpy as jnp
from jax import lax
from jax.experimental import pallas as pl
from jax.experimental.pallas import tpu as pltpu
from jax.experimental.pallas import tpu_sc as plsc

_PI = 3.141592653589793

NUM_ENT = 1000000
NUM_REL = 1000
D = 64
B = 16384
NC = 2
NS = 16
NW = NC * NS
BPW = B // NW
CHUNK = 128
NCHUNK = BPW // CHUNK
NGRP = CHUNK // 16

WPT = 245
RANGE = WPT * 128
GW = 5
NCH = WPT // GW
CW = GW * 128
TAIL0 = (NUM_ENT // 128) * 128
CAP = 1408
SENT = 0x7FFFFFF0


def _cos_body(rel_ref, out_ref):
    out_ref[...] = jnp.cos(rel_ref[...] * jnp.float32(_PI))


def _cos_table(rel_table):
    return pl.pallas_call(
        _cos_body,
        out_shape=jax.ShapeDtypeStruct((NUM_REL, D), jnp.float32),
    )(rel_table)


def _scal(v, j):
    return lax.squeeze(lax.slice(v, (j,), (j + 1,)), (0,))


_GATHER_DNUMS = lax.GatherDimensionNumbers(
    offset_dims=(), collapsed_slice_dims=(0,), start_index_map=(0,))


def _rotate16(v, sh):
    idx = (lax.iota(jnp.int32, 16) + sh) & 15
    return lax.gather(v, idx[:, None], _GATHER_DNUMS, (1,),
                      indices_are_sorted=False, unique_indices=False,
                      mode=lax.GatherScatterMode.PROMISE_IN_BOUNDS)


def _hsum16(v):
    for sh in (8, 4, 2, 1):
        v = v + _rotate16(v, sh)
    return v


def _sqrt16(x):
    i = lax.bitcast_convert_type(x, jnp.int32)
    y = lax.bitcast_convert_type(jnp.int32(0x5F3759DF) - (i >> 1), jnp.float32)
    xh = 0.5 * x
    for _ in range(3):
        y = y * (1.5 - xh * y * y)
    return x * y


_MESH = plsc.VectorSubcoreMesh(core_axis_name="c", subcore_axis_name="s")


@functools.partial(
    pl.kernel,
    mesh=_MESH,
    compiler_params=pltpu.CompilerParams(use_tc_tiling_on_sc=True,
                                         needs_layout_passes=False),
    out_type=(
        jax.ShapeDtypeStruct((NW * CAP, D), jnp.float32),
        jax.ShapeDtypeStruct((NW * CAP,), jnp.int32),
    ),
    scratch_types=[
        pltpu.VMEM((B,), jnp.int32),
        pltpu.VMEM((CAP + 16,), jnp.int32),
        pltpu.VMEM((CAP + 16,), jnp.int32),
        pltpu.VMEM((CAP + 16,), jnp.int32),
        pltpu.VMEM((CAP + 16,), jnp.int32),
        pltpu.VMEM((CAP,), jnp.int32),
        pltpu.VMEM((2, D, CW), jnp.float32),
        pltpu.VMEM((16, D), jnp.float32),
        pltpu.VMEM((D,), jnp.float32),
        pltpu.SemaphoreType.DMA,
        pltpu.SemaphoreType.DMA,
    ],
)
def _sc_stream(ent_hbm, hidx_hbm, tidx_hbm, rows_hbm, slots_hbm,
               idx_v, le_v, ls_v, ce_v, cs_v, so_v, chunk_v, ring_v, dummy_v, sem, sem2):
    wid = lax.axis_index("s") * NC + lax.axis_index("c")
    lo = wid * RANGE
    hi = jnp.minimum(lo + RANGE, TAIL0)
    lanes = lax.iota(jnp.int32, 16)

    def fill(i, c):
        so_v[pl.ds(i * 16, 16)] = jnp.full((16,), SENT, jnp.int32)
        return c
    lax.fori_loop(0, CAP // 16, fill, 0)

    def scan_table(base_slot, cnt0):
        def scan(i, cnt):
            v = idx_v[pl.ds(i * 16, 16)]
            m = (v >= lo) & (v < hi)
            pos = base_slot + i * 16 + lanes
            c = jnp.minimum(cnt, CAP)
            plsc.store_compressed(le_v.at[pl.ds(c, 16)], v, mask=m)
            plsc.store_compressed(ls_v.at[pl.ds(c, 16)], pos, mask=m)
            npc = plsc.all_reduce_population_count(m)
            return cnt + _scal(npc, 0)
        return lax.fori_loop(0, B // 16, scan, cnt0)

    pltpu.sync_copy(hidx_hbm, idx_v)
    cnt = scan_table(0, jnp.int32(0))
    pltpu.sync_copy(tidx_hbm, idx_v)
    cnt = scan_table(B, cnt)
    cnt = jnp.minimum(cnt, CAP)
    nvec = (cnt + 15) >> 4

    last_full = (NUM_ENT - CW) // 128

    def process(app, lo_c, hi_c, off, width, buf):
        def match(i, mcnt):
            v = le_v[pl.ds(i * 16, 16)]
            s = ls_v[pl.ds(i * 16, 16)]
            m = (v >= lo_c) & (v < hi_c)
            mc = jnp.minimum(mcnt, CAP)
            plsc.store_compressed(ce_v.at[pl.ds(mc, 16)], v - off, mask=m)
            plsc.store_compressed(cs_v.at[pl.ds(mc, 16)], s, mask=m)
            npc = plsc.all_reduce_population_count(m)
            return mcnt + _scal(npc, 0)
        mcnt = lax.fori_loop(0, nvec, match, jnp.int32(0))
        mcnt = jnp.minimum(mcnt, CAP - app)

        def drain1(j2, c3):
            pltpu.make_async_copy(rows_hbm.at[0], dummy_v, sem).wait()
            return c3

        def emit(j, c2):
            @pl.when(j >= 16)
            def _():
                drain1(0, 0)
            col = _scal(ce_v[pl.ds(j, 16)], 0)
            colv = lanes * 0 + col
            rv = j & 15
            for g in range(D // 16):
                vals = plsc.load_gather(chunk_v.at[buf], [g * 16 + lanes, colv])
                ring_v[rv, pl.ds(g * 16, 16)] = vals
            pltpu.async_copy(ring_v.at[rv],
                             rows_hbm.at[wid * CAP + app + j], sem)
            return c2
        lax.fori_loop(0, mcnt, emit, 0)

        def cpy(m, c2):
            v = cs_v[pl.ds(m * 16, 16)]
            rem = mcnt - m * 16
            pm = lanes < rem
            plsc.store_compressed(so_v.at[pl.ds(app + m * 16, 16)], v, mask=pm)
            return c2
        lax.fori_loop(0, (mcnt + 15) >> 4, cpy, 0)

        lax.fori_loop(0, jnp.minimum(mcnt, 16), drain1, 0)

        return app + mcnt

    def start_fetch(c):
        lo_c = lo + c * CW
        off = 128 * jnp.minimum((lo_c >> 7), last_full)
        off = pl.multiple_of(off, 128)
        buf = c & 1
        pltpu.async_copy(ent_hbm.at[:, pl.ds(off, CW)], chunk_v.at[buf], sem2)
        return off

    def wait_fetch(c):
        pltpu.make_async_copy(ent_hbm.at[pl.ds(0, D), pl.ds(0, CW)],
                              chunk_v.at[c & 1], sem2).wait()

    start_fetch(jnp.int32(0))

    def chunk_step(c, carry):
        lo_c = lo + c * CW
        off = 128 * jnp.minimum((lo_c >> 7), last_full)
        wait_fetch(c)

        @pl.when(c + 1 < NCH)
        def _():
            start_fetch(c + 1)

        hi_c = jnp.minimum(lo_c + CW, TAIL0)
        return process(carry, lo_c, hi_c, off, CW, c & 1)

    lax.fori_loop(0, NCH, chunk_step, jnp.int32(0))


    pltpu.sync_copy(so_v, slots_hbm.at[pl.ds(wid * CAP, CAP)])


@functools.partial(
    pl.kernel,
    mesh=_MESH,
    compiler_params=pltpu.CompilerParams(use_tc_tiling_on_sc=True,
                                         needs_layout_passes=False),
    out_type=jax.ShapeDtypeStruct((2, B), jnp.int32),
    scratch_types=[
        pltpu.VMEM((NW * CAP,), jnp.int32),
        pltpu.VMEM((BPW + 16,), jnp.int32),
        pltpu.VMEM((BPW + 16,), jnp.int32),
    ],
)
def _sc_invert(slots_hbm, loc_hbm, sl_v, lh_v, lt_v):
    wid = lax.axis_index("s") * NC + lax.axis_index("c")
    base = pl.multiple_of(wid * BPW, BPW)
    lanes = lax.iota(jnp.int32, 16)

    pltpu.sync_copy(slots_hbm, sl_v)

    def fill(i, c):
        lh_v[pl.ds(i * 16, 16)] = jnp.full((16,), -1, jnp.int32)
        lt_v[pl.ds(i * 16, 16)] = jnp.full((16,), -1, jnp.int32)
        return c
    lax.fori_loop(0, BPW // 16, fill, 0)

    def inv(i, c):
        v = sl_v[pl.ds(i * 16, 16)]
        pos = i * 16 + lanes
        rh = (v - base) & (BPW - 1)
        mh = (v >= base) & (v < base + BPW)
        plsc.store_scatter(lh_v, [rh], pos, mask=mh)
        rt = (v - (B + base)) & (BPW - 1)
        mt = (v >= B + base) & (v < B + base + BPW)
        plsc.store_scatter(lt_v, [rt], pos, mask=mt)
        return c
    lax.fori_loop(0, (NW * CAP) // 16, inv, 0)

    pltpu.sync_copy(lh_v.at[pl.ds(0, BPW)], loc_hbm.at[0, pl.ds(base, BPW)])
    pltpu.sync_copy(lt_v.at[pl.ds(0, BPW)], loc_hbm.at[1, pl.ds(base, BPW)])


@functools.partial(
    pl.kernel,
    mesh=_MESH,
    compiler_params=pltpu.CompilerParams(use_tc_tiling_on_sc=True),
    out_type=jax.ShapeDtypeStruct((B,), jnp.float32),
    scratch_types=[
        pltpu.VMEM((BPW,), jnp.int32),
        pltpu.VMEM((BPW,), jnp.int32),
        pltpu.VMEM((BPW,), jnp.int32),
        pltpu.VMEM((BPW,), jnp.int32),
        pltpu.VMEM((BPW,), jnp.int32),
        pltpu.VMEM((CHUNK, D), jnp.float32),
        pltpu.VMEM((CHUNK, D), jnp.float32),
        pltpu.VMEM((CHUNK, D), jnp.float32),
        pltpu.VMEM((BPW,), jnp.float32),
        pltpu.SemaphoreType.DMA,
    ],
)
def _sc_compute(rows_hbm, loc_hbm, cos_hbm, ridx_hbm, tail_hbm,
                hidx_hbm, tidx_hbm, out_hbm,
                lh_v, lt_v, ridx_v, hidx_v, tidx_v,
                h_rows, c_rows, t_rows, out_v, sem):
    wid = lax.axis_index("s") * NC + lax.axis_index("c")
    base = pl.multiple_of(wid * BPW, BPW)
    lanes = lax.iota(jnp.int32, 16)

    pltpu.sync_copy(loc_hbm.at[0, pl.ds(base, BPW)], lh_v)
    pltpu.sync_copy(loc_hbm.at[1, pl.ds(base, BPW)], lt_v)
    pltpu.sync_copy(ridx_hbm.at[pl.ds(base, BPW)], ridx_v)
    pltpu.sync_copy(hidx_hbm.at[pl.ds(base, BPW)], hidx_v)
    pltpu.sync_copy(tidx_hbm.at[pl.ds(base, BPW)], tidx_v)

    def drain_group(row0):
        sl = pl.ds(row0, 16)
        pltpu.make_async_copy(cos_hbm.at[pl.ds(0, 16)], h_rows.at[sl], sem).wait()
        pltpu.make_async_copy(cos_hbm.at[pl.ds(0, 16)], t_rows.at[sl], sem).wait()
        pltpu.make_async_copy(cos_hbm.at[pl.ds(0, 16)], c_rows.at[sl], sem).wait()

    for k in range(NCHUNK):
        def fetch(g, carry):
            row0 = g * 16
            sl = pl.ds(k * CHUNK + row0, 16)
            hv = lh_v[sl]
            tv = lt_v[sl]
            rv = ridx_v[sl]
            hev = hidx_v[sl]
            tev = tidx_v[sl]
            for rr in range(16):
                lh = _scal(hv, rr)
                lt = _scal(tv, rr)

                @pl.when(lh >= 0)
                def _():
                    pltpu.async_copy(rows_hbm.at[lh],
                                     h_rows.at[row0 + rr], sem)

                @pl.when(lh < 0)
                def _():
                    pltpu.async_copy(tail_hbm.at[_scal(hev, rr) - TAIL0],
                                     h_rows.at[row0 + rr], sem)

                @pl.when(lt >= 0)
                def _():
                    pltpu.async_copy(rows_hbm.at[lt],
                                     t_rows.at[row0 + rr], sem)

                @pl.when(lt < 0)
                def _():
                    pltpu.async_copy(tail_hbm.at[_scal(tev, rr) - TAIL0],
                                     t_rows.at[row0 + rr], sem)

                pltpu.async_copy(cos_hbm.at[_scal(rv, rr)],
                                 c_rows.at[row0 + rr], sem)

            @pl.when(g > 0)
            def _():
                drain_group(row0 - 16)

            return carry

        lax.fori_loop(0, NGRP, fetch, 0)
        drain_group(CHUNK - 16)

        def group(g, carry):
            row0 = g * 16
            ov = jnp.zeros((16,), jnp.float32)
            for rr in range(16):
                i = row0 + rr
                acc = jnp.zeros((16,), jnp.float32)
                for j in range(D // 16):
                    sl = pl.ds(j * 16, 16)
                    hv = h_rows[i, sl]
                    tv = t_rows[i, sl]
                    cv = c_rows[i, sl]
                    x = hv * hv + tv * tv - 2.0 * (hv * tv) * cv
                    acc = acc + _sqrt16(jnp.maximum(x, 0.0))
                ov = jnp.where(lanes == rr, -_hsum16(acc), ov)
            out_v[pl.ds(k * CHUNK + row0, 16)] = ov
            return carry

        lax.fori_loop(0, NGRP, group, 0)

    pltpu.sync_copy(out_v, out_hbm.at[pl.ds(base, BPW)])


def kernel(h_idx, r_idx, t_idx, ent_table, rel_table):
    cos_table = _cos_table(rel_table)
    h1 = h_idx.astype(jnp.int32)
    r1 = r_idx.astype(jnp.int32)
    t1 = t_idx.astype(jnp.int32)
    rows, slots = _sc_stream(ent_table.T, h1, t1)
    locs = _sc_invert(slots)
    tail_rm = ent_table[TAIL0:]
    return _sc_compute(rows, locs, cos_table, r1, tail_rm, h1, t1)


# speedup vs baseline: 1.0270x; 1.0270x over previous
import functools

import jax
import jax.numpy as jnp
from jax import lax
from jax.experimental import pallas as pl
from jax.experimental.pallas import tpu as pltpu
from jax.experimental.pallas import tpu_sc as plsc

_PI = 3.141592653589793

NUM_ENT = 1000000
NUM_REL = 1000
D = 64
B = 16384
NC = 2
NS = 16
NW = NC * NS
BPW = B // NW
CHUNK = 128
NCHUNK = BPW // CHUNK
NGRP = CHUNK // 16

WPT = 245
RANGE = WPT * 128
GW = 5
NCH = WPT // GW
CW = GW * 128
TAIL0 = (NUM_ENT // 128) * 128
CAP = 1408
SENT = 0x7FFFFFF0


def _cos_body(rel_ref, out_ref):
    out_ref[...] = jnp.cos(rel_ref[...] * jnp.float32(_PI))


def _cos_table(rel_table):
    return pl.pallas_call(
        _cos_body,
        out_shape=jax.ShapeDtypeStruct((NUM_REL, D), jnp.float32),
    )(rel_table)


def _scal(v, j):
    return lax.squeeze(lax.slice(v, (j,), (j + 1,)), (0,))


_GATHER_DNUMS = lax.GatherDimensionNumbers(
    offset_dims=(), collapsed_slice_dims=(0,), start_index_map=(0,))


def _rotate16(v, sh):
    idx = (lax.iota(jnp.int32, 16) + sh) & 15
    return lax.gather(v, idx[:, None], _GATHER_DNUMS, (1,),
                      indices_are_sorted=False, unique_indices=False,
                      mode=lax.GatherScatterMode.PROMISE_IN_BOUNDS)


def _hsum16(v):
    for sh in (8, 4, 2, 1):
        v = v + _rotate16(v, sh)
    return v


def _sqrt16(x):
    i = lax.bitcast_convert_type(x, jnp.int32)
    y = lax.bitcast_convert_type(jnp.int32(0x5F3759DF) - (i >> 1), jnp.float32)
    xh = 0.5 * x
    for _ in range(3):
        y = y * (1.5 - xh * y * y)
    return x * y


_MESH = plsc.VectorSubcoreMesh(core_axis_name="c", subcore_axis_name="s")


@functools.partial(
    pl.kernel,
    mesh=_MESH,
    compiler_params=pltpu.CompilerParams(use_tc_tiling_on_sc=True,
                                         needs_layout_passes=False),
    out_type=(
        jax.ShapeDtypeStruct((NW * CAP, D), jnp.float32),
        jax.ShapeDtypeStruct((NW * CAP,), jnp.int32),
        jax.ShapeDtypeStruct((NW * 16,), jnp.int32),
    ),
    scratch_types=[
        pltpu.VMEM((B,), jnp.int32),
        pltpu.VMEM((CAP + 16,), jnp.int32),
        pltpu.VMEM((CAP + 16,), jnp.int32),
        pltpu.VMEM((CAP + 16,), jnp.int32),
        pltpu.VMEM((CAP + 16,), jnp.int32),
        pltpu.VMEM((CAP,), jnp.int32),
        pltpu.VMEM((2, D, CW), jnp.float32),
        pltpu.VMEM((16, D), jnp.float32),
        pltpu.VMEM((D,), jnp.float32),
        pltpu.SemaphoreType.DMA,
        pltpu.SemaphoreType.DMA,
    ],
)
def _sc_stream(ent_hbm, hidx_hbm, tidx_hbm, rows_hbm, slots_hbm, cnt_hbm,
               idx_v, le_v, ls_v, ce_v, cs_v, so_v, chunk_v, ring_v, dummy_v, sem, sem2):
    wid = lax.axis_index("s") * NC + lax.axis_index("c")
    lo = wid * RANGE
    hi = jnp.minimum(lo + RANGE, TAIL0)
    lanes = lax.iota(jnp.int32, 16)

    def fill(i, c):
        so_v[pl.ds(i * 16, 16)] = jnp.full((16,), SENT, jnp.int32)
        return c
    lax.fori_loop(0, CAP // 16, fill, 0)

    def scan_table(base_slot, cnt0):
        def scan(i, cnt):
            v = idx_v[pl.ds(i * 16, 16)]
            m = (v >= lo) & (v < hi)
            pos = base_slot + i * 16 + lanes
            c = jnp.minimum(cnt, CAP)
            plsc.store_compressed(le_v.at[pl.ds(c, 16)], v, mask=m)
            plsc.store_compressed(ls_v.at[pl.ds(c, 16)], pos, mask=m)
            npc = plsc.all_reduce_population_count(m)
            return cnt + _scal(npc, 0)
        return lax.fori_loop(0, B // 16, scan, cnt0)

    pltpu.sync_copy(hidx_hbm, idx_v)
    cnt = scan_table(0, jnp.int32(0))
    pltpu.sync_copy(tidx_hbm, idx_v)
    cnt = scan_table(B, cnt)
    cnt = jnp.minimum(cnt, CAP)
    nvec = (cnt + 15) >> 4

    last_full = (NUM_ENT - CW) // 128

    def process(app, lo_c, hi_c, off, width, buf):
        def match(i, mcnt):
            v = le_v[pl.ds(i * 16, 16)]
            s = ls_v[pl.ds(i * 16, 16)]
            m = (v >= lo_c) & (v < hi_c)
            mc = jnp.minimum(mcnt, CAP)
            plsc.store_compressed(ce_v.at[pl.ds(mc, 16)], v - off, mask=m)
            plsc.store_compressed(cs_v.at[pl.ds(mc, 16)], s, mask=m)
            npc = plsc.all_reduce_population_count(m)
            return mcnt + _scal(npc, 0)
        mcnt = lax.fori_loop(0, nvec, match, jnp.int32(0))
        mcnt = jnp.minimum(mcnt, CAP - app)

        def drain1(j2, c3):
            pltpu.make_async_copy(rows_hbm.at[0], dummy_v, sem).wait()
            return c3

        def emit(j, c2):
            @pl.when(j >= 16)
            def _():
                drain1(0, 0)
            col = _scal(ce_v[pl.ds(j, 16)], 0)
            colv = lanes * 0 + col
            rv = j & 15
            for g in range(D // 16):
                vals = plsc.load_gather(chunk_v.at[buf], [g * 16 + lanes, colv])
                ring_v[rv, pl.ds(g * 16, 16)] = vals
            pltpu.async_copy(ring_v.at[rv],
                             rows_hbm.at[wid * CAP + app + j], sem)
            return c2
        lax.fori_loop(0, mcnt, emit, 0)

        def cpy(m, c2):
            v = cs_v[pl.ds(m * 16, 16)]
            rem = mcnt - m * 16
            pm = lanes < rem
            plsc.store_compressed(so_v.at[pl.ds(app + m * 16, 16)], v, mask=pm)
            return c2
        lax.fori_loop(0, (mcnt + 15) >> 4, cpy, 0)

        lax.fori_loop(0, jnp.minimum(mcnt, 16), drain1, 0)

        return app + mcnt

    def start_fetch(c):
        lo_c = lo + c * CW
        off = 128 * jnp.minimum((lo_c >> 7), last_full)
        off = pl.multiple_of(off, 128)
        buf = c & 1
        pltpu.async_copy(ent_hbm.at[:, pl.ds(off, CW)], chunk_v.at[buf], sem2)
        return off

    def wait_fetch(c):
        pltpu.make_async_copy(ent_hbm.at[pl.ds(0, D), pl.ds(0, CW)],
                              chunk_v.at[c & 1], sem2).wait()

    start_fetch(jnp.int32(0))

    def chunk_step(c, carry):
        lo_c = lo + c * CW
        off = 128 * jnp.minimum((lo_c >> 7), last_full)
        wait_fetch(c)

        @pl.when(c + 1 < NCH)
        def _():
            start_fetch(c + 1)

        hi_c = jnp.minimum(lo_c + CW, TAIL0)
        return process(carry, lo_c, hi_c, off, CW, c & 1)

    app_f = lax.fori_loop(0, NCH, chunk_step, jnp.int32(0))

    pltpu.sync_copy(so_v, slots_hbm.at[pl.ds(wid * CAP, CAP)])
    cs_v[pl.ds(0, 16)] = lanes * 0 + app_f
    pltpu.sync_copy(cs_v.at[pl.ds(0, 16)], cnt_hbm.at[pl.ds(wid * 16, 16)])


@functools.partial(
    pl.kernel,
    mesh=_MESH,
    compiler_params=pltpu.CompilerParams(use_tc_tiling_on_sc=True,
                                         needs_layout_passes=False),
    out_type=jax.ShapeDtypeStruct((2, B), jnp.int32),
    scratch_types=[
        pltpu.VMEM((NW * CAP,), jnp.int32),
        pltpu.VMEM((NW * 16,), jnp.int32),
        pltpu.VMEM((BPW + 16,), jnp.int32),
        pltpu.VMEM((BPW + 16,), jnp.int32),
    ],
)
def _sc_invert(slots_hbm, cnt_hbm, loc_hbm, sl_v, cnt_v, lh_v, lt_v):
    wid = lax.axis_index("s") * NC + lax.axis_index("c")
    base = pl.multiple_of(wid * BPW, BPW)
    lanes = lax.iota(jnp.int32, 16)

    pltpu.sync_copy(slots_hbm, sl_v)
    pltpu.sync_copy(cnt_hbm, cnt_v)

    def fill(i, c):
        lh_v[pl.ds(i * 16, 16)] = jnp.full((16,), -1, jnp.int32)
        lt_v[pl.ds(i * 16, 16)] = jnp.full((16,), -1, jnp.int32)
        return c
    lax.fori_loop(0, BPW // 16, fill, 0)

    def region(r, c):
        cr = _scal(cnt_v[pl.ds(r * 16, 16)], 0)
        rbase = r * CAP

        def inv(i, c2):
            pos = rbase + i * 16
            v = sl_v[pl.ds(pos, 16)]
            posv = pos + lanes
            rh = (v - base) & (BPW - 1)
            mh = (v >= base) & (v < base + BPW)
            plsc.store_scatter(lh_v, [rh], posv, mask=mh)
            rt = (v - (B + base)) & (BPW - 1)
            mt = (v >= B + base) & (v < B + base + BPW)
            plsc.store_scatter(lt_v, [rt], posv, mask=mt)
            return c2
        lax.fori_loop(0, (cr + 15) >> 4, inv, 0)
        return c
    lax.fori_loop(0, NW, region, 0)

    pltpu.sync_copy(lh_v.at[pl.ds(0, BPW)], loc_hbm.at[0, pl.ds(base, BPW)])
    pltpu.sync_copy(lt_v.at[pl.ds(0, BPW)], loc_hbm.at[1, pl.ds(base, BPW)])


@functools.partial(
    pl.kernel,
    mesh=_MESH,
    compiler_params=pltpu.CompilerParams(use_tc_tiling_on_sc=True),
    out_type=jax.ShapeDtypeStruct((B,), jnp.float32),
    scratch_types=[
        pltpu.VMEM((BPW,), jnp.int32),
        pltpu.VMEM((BPW,), jnp.int32),
        pltpu.VMEM((BPW,), jnp.int32),
        pltpu.VMEM((BPW,), jnp.int32),
        pltpu.VMEM((BPW,), jnp.int32),
        pltpu.VMEM((CHUNK, D), jnp.float32),
        pltpu.VMEM((CHUNK, D), jnp.float32),
        pltpu.VMEM((CHUNK, D), jnp.float32),
        pltpu.VMEM((BPW,), jnp.float32),
        pltpu.SemaphoreType.DMA,
    ],
)
def _sc_compute(rows_hbm, loc_hbm, cos_hbm, ridx_hbm, tail_hbm,
                hidx_hbm, tidx_hbm, out_hbm,
                lh_v, lt_v, ridx_v, hidx_v, tidx_v,
                h_rows, c_rows, t_rows, out_v, sem):
    wid = lax.axis_index("s") * NC + lax.axis_index("c")
    base = pl.multiple_of(wid * BPW, BPW)
    lanes = lax.iota(jnp.int32, 16)

    pltpu.sync_copy(loc_hbm.at[0, pl.ds(base, BPW)], lh_v)
    pltpu.sync_copy(loc_hbm.at[1, pl.ds(base, BPW)], lt_v)
    pltpu.sync_copy(ridx_hbm.at[pl.ds(base, BPW)], ridx_v)
    pltpu.sync_copy(hidx_hbm.at[pl.ds(base, BPW)], hidx_v)
    pltpu.sync_copy(tidx_hbm.at[pl.ds(base, BPW)], tidx_v)

    def drain_group(row0):
        sl = pl.ds(row0, 16)
        pltpu.make_async_copy(cos_hbm.at[pl.ds(0, 16)], h_rows.at[sl], sem).wait()
        pltpu.make_async_copy(cos_hbm.at[pl.ds(0, 16)], t_rows.at[sl], sem).wait()
        pltpu.make_async_copy(cos_hbm.at[pl.ds(0, 16)], c_rows.at[sl], sem).wait()

    for k in range(NCHUNK):
        def fetch(g, carry):
            row0 = g * 16
            sl = pl.ds(k * CHUNK + row0, 16)
            hv = lh_v[sl]
            tv = lt_v[sl]
            rv = ridx_v[sl]
            hev = hidx_v[sl]
            tev = tidx_v[sl]
            for rr in range(16):
                lh = _scal(hv, rr)
                lt = _scal(tv, rr)

                @pl.when(lh >= 0)
                def _():
                    pltpu.async_copy(rows_hbm.at[lh],
                                     h_rows.at[row0 + rr], sem)

                @pl.when(lh < 0)
                def _():
                    pltpu.async_copy(tail_hbm.at[_scal(hev, rr) - TAIL0],
                                     h_rows.at[row0 + rr], sem)

                @pl.when(lt >= 0)
                def _():
                    pltpu.async_copy(rows_hbm.at[lt],
                                     t_rows.at[row0 + rr], sem)

                @pl.when(lt < 0)
                def _():
                    pltpu.async_copy(tail_hbm.at[_scal(tev, rr) - TAIL0],
                                     t_rows.at[row0 + rr], sem)

                pltpu.async_copy(cos_hbm.at[_scal(rv, rr)],
                                 c_rows.at[row0 + rr], sem)

            @pl.when(g > 0)
            def _():
                drain_group(row0 - 16)

            return carry

        lax.fori_loop(0, NGRP, fetch, 0)
        drain_group(CHUNK - 16)

        def group(g, carry):
            row0 = g * 16
            ov = jnp.zeros((16,), jnp.float32)
            for rr in range(16):
                i = row0 + rr
                acc = jnp.zeros((16,), jnp.float32)
                for j in range(D // 16):
                    sl = pl.ds(j * 16, 16)
                    hv = h_rows[i, sl]
                    tv = t_rows[i, sl]
                    cv = c_rows[i, sl]
                    x = hv * hv + tv * tv - 2.0 * (hv * tv) * cv
                    acc = acc + _sqrt16(jnp.maximum(x, 0.0))
                ov = jnp.where(lanes == rr, -_hsum16(acc), ov)
            out_v[pl.ds(k * CHUNK + row0, 16)] = ov
            return carry

        lax.fori_loop(0, NGRP, group, 0)

    pltpu.sync_copy(out_v, out_hbm.at[pl.ds(base, BPW)])


def kernel(h_idx, r_idx, t_idx, ent_table, rel_table):
    cos_table = _cos_table(rel_table)
    h1 = h_idx.astype(jnp.int32)
    r1 = r_idx.astype(jnp.int32)
    t1 = t_idx.astype(jnp.int32)
    rows, slots, cnts = _sc_stream(ent_table.T, h1, t1)
    locs = _sc_invert(slots, cnts)
    tail_rm = ent_table[TAIL0:]
    return _sc_compute(rows, locs, cos_table, r1, tail_rm, h1, t1)
